# baseline (speedup 1.0000x reference)
import functools

import jax
import jax.numpy as jnp
from jax import lax
from jax.experimental import pallas as pl
from jax.experimental.pallas import tpu as pltpu
from jax.experimental.pallas import tpu_sc as plsc

NUM_EMB = 1024
DIM = 64
N_TOKENS = 16 * 64 * 64
BN = 4096
N_CHUNKS = 1
CHUNK = N_TOKENS // N_CHUNKS
GRID = CHUNK // BN

NC, NS = 2, 16
NW = NC * NS
B_PER_W = CHUNK // NW
CH = 128


def _dist_argmin_body(x_ref, w_ref, idx_ref, w2_ref, iotaf_ref, wm2_ref):
    @pl.when(pl.program_id(0) == 0)
    def _():
        w = w_ref[...]
        w2_ref[...] = jnp.sum(w * w, axis=1)[None, :]
        ii = lax.broadcasted_iota(jnp.int32, (1, NUM_EMB), 1)
        iotaf_ref[...] = ii.astype(jnp.float32)
        wm2_ref[...] = w * -2.0

    x = jnp.transpose(x_ref[0])
    x2 = jnp.sum(x * x, axis=1, keepdims=True)
    mm2 = lax.dot_general(x, wm2_ref[...], (((1,), (1,)), ((), ())),
                          preferred_element_type=jnp.float32)
    d = (x2 + w2_ref[...]) + mm2
    dmin = jnp.min(d, axis=1, keepdims=True)
    idxf = jnp.min(jnp.where(d == dmin, iotaf_ref[...], float(2 * NUM_EMB)),
                   axis=1)
    idx_ref[0, 0, :] = idxf.astype(jnp.int32)


def _encode(x_cm, W):
    sb = 4096 // BN
    return pl.pallas_call(
        _dist_argmin_body,
        grid=(GRID,),
        in_specs=[
            pl.BlockSpec((1, DIM, BN), lambda i: (i // sb, 0, i % sb)),
            pl.BlockSpec((NUM_EMB, DIM), lambda i: (0, 0)),
        ],
        out_specs=pl.BlockSpec((1, 1, BN), lambda i: (i, 0, 0)),
        out_shape=jax.ShapeDtypeStruct((GRID, 1, BN), jnp.int32),
        scratch_shapes=[pltpu.VMEM((1, NUM_EMB), jnp.float32),
                        pltpu.VMEM((1, NUM_EMB), jnp.float32),
                        pltpu.VMEM((NUM_EMB, DIM), jnp.float32)],
    )(x_cm, W)


D2 = 2 * DIM


@functools.partial(
    pl.kernel,
    out_type=jax.ShapeDtypeStruct((CHUNK, D2), jnp.float32),
    mesh=plsc.VectorSubcoreMesh(core_axis_name="c", subcore_axis_name="s"),
    scratch_types=[
        pltpu.VMEM((B_PER_W,), jnp.int32),
        pltpu.VMEM((CH, D2), jnp.float32),
        pltpu.SemaphoreType.DMA,
    ],
)
def _gather_rows(w_hbm, idx_hbm, out_hbm, idx_v, rows_v, sem):
    wid = lax.axis_index("s") * NC + lax.axis_index("c")
    base = wid * B_PER_W
    pltpu.sync_copy(idx_hbm.at[pl.ds(base, B_PER_W)], idx_v)
    for c in range(B_PER_W // CH):
        pltpu.async_copy(
            w_hbm.at[idx_v.at[pl.ds(c * CH, CH)]], rows_v, sem).wait()
        pltpu.sync_copy(rows_v, out_hbm.at[pl.ds(base + c * CH, CH)])


def _transpose_body(q_ref, o_ref):
    o_ref[0] = jnp.transpose(q_ref[0][:, :DIM])


def _transpose_out(q128):
    return pl.pallas_call(
        _transpose_body,
        grid=(16,),
        in_specs=[pl.BlockSpec((1, 4096, D2), lambda i: (i, 0, 0))],
        out_specs=pl.BlockSpec((1, DIM, 4096), lambda i: (i, 0, 0)),
        out_shape=jax.ShapeDtypeStruct((16, DIM, 4096), jnp.float32),
    )(q128)


def kernel(inputs, W):
    b, c, h, w = inputs.shape
    x_cm = inputs.reshape(b, c, h * w)
    idx = _encode(x_cm, W).reshape(-1)
    wd = jnp.concatenate([W, W], axis=1)
    q128 = _gather_rows(wd, idx)
    quantized = _transpose_out(q128.reshape(b, h * w, D2)).reshape(
        b, c, h, w)
    return quantized, idx.reshape(b, h, w)


# speedup vs baseline: 1.2314x; 1.2314x over previous
import functools

import jax
import jax.numpy as jnp
from jax import lax
from jax.experimental import pallas as pl
from jax.experimental.pallas import tpu as pltpu
from jax.experimental.pallas import tpu_sc as plsc

NUM_EMB = 1024
DIM = 64
N_TOKENS = 16 * 64 * 64
BN = 4096
N_CHUNKS = 1
CHUNK = N_TOKENS // N_CHUNKS
GRID = CHUNK // BN

NC, NS = 2, 16
NW = NC * NS
B_PER_W = CHUNK // NW
CH = 128


def _dist_argmin_body(x_ref, w_ref, idx_ref, w2_ref, iotaf_ref, wm2_ref):
    @pl.when(pl.program_id(0) == 0)
    def _():
        w = w_ref[...]
        w2_ref[...] = jnp.sum(w * w, axis=1)[None, :]
        ii = lax.broadcasted_iota(jnp.int32, (1, NUM_EMB), 1)
        iotaf_ref[...] = ii.astype(jnp.float32)
        wm2_ref[...] = w * -2.0

    x = jnp.transpose(x_ref[0].reshape(DIM, BN))
    x2 = jnp.sum(x * x, axis=1, keepdims=True)
    mm2 = lax.dot_general(x, wm2_ref[...], (((1,), (1,)), ((), ())),
                          preferred_element_type=jnp.float32)
    d = (x2 + w2_ref[...]) + mm2
    dmin = jnp.min(d, axis=1, keepdims=True)
    idxf = jnp.min(jnp.where(d == dmin, iotaf_ref[...], float(2 * NUM_EMB)),
                   axis=1)
    idx_ref[0, 0, :] = idxf.astype(jnp.int32)


def _encode(x4d, W):
    return pl.pallas_call(
        _dist_argmin_body,
        grid=(GRID,),
        in_specs=[
            pl.BlockSpec((1, DIM, 64, 64), lambda i: (i, 0, 0, 0)),
            pl.BlockSpec((NUM_EMB, DIM), lambda i: (0, 0)),
        ],
        out_specs=pl.BlockSpec((1, 1, BN), lambda i: (i, 0, 0)),
        out_shape=jax.ShapeDtypeStruct((GRID, 1, BN), jnp.int32),
        scratch_shapes=[pltpu.VMEM((1, NUM_EMB), jnp.float32),
                        pltpu.VMEM((1, NUM_EMB), jnp.float32),
                        pltpu.VMEM((NUM_EMB, DIM), jnp.float32)],
    )(x4d, W)


D2 = 2 * DIM


@functools.partial(
    pl.kernel,
    out_type=jax.ShapeDtypeStruct((CHUNK, D2), jnp.float32),
    mesh=plsc.VectorSubcoreMesh(core_axis_name="c", subcore_axis_name="s"),
    scratch_types=[
        pltpu.VMEM((B_PER_W,), jnp.int32),
        pltpu.VMEM((CH, D2), jnp.float32),
        pltpu.SemaphoreType.DMA,
    ],
)
def _gather_rows(w_hbm, idx_hbm, out_hbm, idx_v, rows_v, sem):
    wid = lax.axis_index("s") * NC + lax.axis_index("c")
    base = wid * B_PER_W
    pltpu.sync_copy(idx_hbm.at[pl.ds(base, B_PER_W)], idx_v)
    for c in range(B_PER_W // CH):
        pltpu.async_copy(
            w_hbm.at[idx_v.at[pl.ds(c * CH, CH)]], rows_v, sem).wait()
        pltpu.sync_copy(rows_v, out_hbm.at[pl.ds(base + c * CH, CH)])


def _transpose_body(q_ref, o_ref):
    t = jnp.transpose(q_ref[0][:, :DIM])
    o_ref[0] = t.reshape(DIM, 64, 64)


def _transpose_out(q128):
    return pl.pallas_call(
        _transpose_body,
        grid=(16,),
        in_specs=[pl.BlockSpec((1, 4096, D2), lambda i: (i, 0, 0))],
        out_specs=pl.BlockSpec((1, DIM, 64, 64), lambda i: (i, 0, 0, 0)),
        out_shape=jax.ShapeDtypeStruct((16, DIM, 64, 64), jnp.float32),
    )(q128)


def kernel(inputs, W):
    b, c, h, w = inputs.shape
    idx = _encode(inputs, W).reshape(-1)
    wd = jnp.concatenate([W, W], axis=1)
    q128 = _gather_rows(wd, idx)
    quantized = _transpose_out(q128.reshape(b, h * w, D2))
    return quantized, idx.reshape(b, h, w)


# speedup vs baseline: 1.2468x; 1.0126x over previous
import functools

import jax
import jax.numpy as jnp
from jax import lax
from jax.experimental import pallas as pl
from jax.experimental.pallas import tpu as pltpu
from jax.experimental.pallas import tpu_sc as plsc

NUM_EMB = 1024
DIM = 64
N_TOKENS = 16 * 64 * 64
BN = 4096
N_CHUNKS = 1
CHUNK = N_TOKENS // N_CHUNKS
GRID = CHUNK // BN

NC, NS = 2, 16
NW = NC * NS
B_PER_W = CHUNK // NW
CH = 128


def _dist_argmin_body(x_ref, w_ref, idx_ref, w2_ref, iotaf_ref, wm2_ref):
    @pl.when(pl.program_id(0) == 0)
    def _():
        w = w_ref[...]
        w2_ref[...] = jnp.sum(w * w, axis=1)[None, :]
        ii = lax.broadcasted_iota(jnp.int32, (1, NUM_EMB), 1)
        iotaf_ref[...] = ii.astype(jnp.float32)
        wm2_ref[...] = w * -2.0

    x = jnp.transpose(x_ref[0].reshape(DIM, BN))
    x2 = jnp.sum(x * x, axis=1, keepdims=True)
    mm2 = lax.dot_general(x, wm2_ref[...], (((1,), (1,)), ((), ())),
                          preferred_element_type=jnp.float32)
    d = (x2 + w2_ref[...]) + mm2
    dmin = jnp.min(d, axis=1, keepdims=True)
    idxf = jnp.min(jnp.where(d == dmin, iotaf_ref[...], float(2 * NUM_EMB)),
                   axis=1)
    idx_ref[0, 0, :] = idxf.astype(jnp.int32)


def _encode(x4d, W):
    return pl.pallas_call(
        _dist_argmin_body,
        grid=(GRID,),
        in_specs=[
            pl.BlockSpec((1, DIM, 64, 64), lambda i: (i, 0, 0, 0)),
            pl.BlockSpec((NUM_EMB, DIM), lambda i: (0, 0)),
        ],
        out_specs=pl.BlockSpec((1, 1, BN), lambda i: (i, 0, 0)),
        out_shape=jax.ShapeDtypeStruct((GRID, 1, BN), jnp.int32),
        scratch_shapes=[pltpu.VMEM((1, NUM_EMB), jnp.float32),
                        pltpu.VMEM((1, NUM_EMB), jnp.float32),
                        pltpu.VMEM((NUM_EMB, DIM), jnp.float32)],
    )(x4d, W)


D2 = 2 * DIM


@functools.partial(
    pl.kernel,
    out_type=jax.ShapeDtypeStruct((CHUNK, D2), jnp.float32),
    mesh=plsc.VectorSubcoreMesh(core_axis_name="c", subcore_axis_name="s"),
    scratch_types=[
        pltpu.VMEM((B_PER_W,), jnp.int32),
        pltpu.VMEM((CH, D2), jnp.float32),
        pltpu.VMEM((CH, D2), jnp.float32),
        pltpu.SemaphoreType.DMA,
        pltpu.SemaphoreType.DMA,
    ],
)
def _gather_rows(w_hbm, idx_hbm, out_hbm, idx_v, rows_v0, rows_v1, s0, s1):
    wid = lax.axis_index("s") * NC + lax.axis_index("c")
    base = wid * B_PER_W
    pltpu.sync_copy(idx_hbm.at[pl.ds(base, B_PER_W)], idx_v)
    bufs, sems = (rows_v0, rows_v1), (s0, s1)
    nch = B_PER_W // CH
    cps = [pltpu.async_copy(w_hbm.at[idx_v.at[pl.ds(0, CH)]], rows_v0, s0),
           None]
    for c in range(nch):
        if c + 1 < nch:
            cps[(c + 1) % 2] = pltpu.async_copy(
                w_hbm.at[idx_v.at[pl.ds((c + 1) * CH, CH)]],
                bufs[(c + 1) % 2], sems[(c + 1) % 2])
        cps[c % 2].wait()
        pltpu.sync_copy(bufs[c % 2], out_hbm.at[pl.ds(base + c * CH, CH)])


def _transpose_body(q_ref, o_ref):
    t = jnp.transpose(q_ref[0][:, :DIM])
    o_ref[0] = t.reshape(DIM, 64, 64)


def _transpose_out(q128):
    return pl.pallas_call(
        _transpose_body,
        grid=(16,),
        in_specs=[pl.BlockSpec((1, 4096, D2), lambda i: (i, 0, 0))],
        out_specs=pl.BlockSpec((1, DIM, 64, 64), lambda i: (i, 0, 0, 0)),
        out_shape=jax.ShapeDtypeStruct((16, DIM, 64, 64), jnp.float32),
    )(q128)


def kernel(inputs, W):
    b, c, h, w = inputs.shape
    idx = _encode(inputs, W).reshape(-1)
    wd = jnp.concatenate([W, W], axis=1)
    q128 = _gather_rows(wd, idx)
    quantized = _transpose_out(q128.reshape(b, h * w, D2))
    return quantized, idx.reshape(b, h, w)
